# initial kernel scaffold (unmeasured)
import jax
import jax.numpy as jnp
from jax import lax
from jax.experimental import pallas as pl
from jax.experimental.pallas import tpu as pltpu

N_DEV = 4


def _partial_matmul(A, Wo):
    M, K = A.shape
    _, N = Wo.shape
    bm, bn = 512, 2048

    def body(a_ref, w_ref, o_ref):
        o_ref[...] = jnp.dot(
            a_ref[...], w_ref[...], preferred_element_type=jnp.float32
        )

    return pl.pallas_call(
        body,
        grid=(M // bm, N // bn),
        in_specs=[
            pl.BlockSpec((bm, K), lambda i, j: (i, 0)),
            pl.BlockSpec((K, bn), lambda i, j: (0, j)),
        ],
        out_specs=pl.BlockSpec((bm, bn), lambda i, j: (i, j)),
        out_shape=jax.ShapeDtypeStruct((M, N), jnp.float32),
    )(A, Wo)


def _ring_reduce_scatter(P, B, S, SB, N):
    n_hops = N_DEV - 1
    TN = 4096

    def body(p_ref, out_ref, recv_ref, va, vb, sem_a, sem_b, sem_o,
             send_sems, recv_sems):
        my = lax.axis_index("i")
        left = lax.rem(my + N_DEV - 1, N_DEV)
        right = lax.rem(my + 1, N_DEV)

        barrier = pltpu.get_barrier_semaphore()
        for nbr in (left, right):
            pl.semaphore_signal(
                barrier, inc=1,
                device_id=(nbr,), device_id_type=pl.DeviceIdType.MESH,
            )
        pl.semaphore_wait(barrier, 2)

        def p_rows(b, bb):
            return pl.ds(bb * S + SB * b, SB)

        for t in range(n_hops):
            b_send = lax.rem(my + (N_DEV - 1 - t), N_DEV)
            rdmas = []
            for bb in range(B):
                if t == 0:
                    src = p_ref.at[p_rows(b_send, bb), :]
                else:
                    src = recv_ref.at[t - 1, pl.ds(bb * SB, SB), :]
                rdma = pltpu.make_async_remote_copy(
                    src_ref=src,
                    dst_ref=recv_ref.at[t, pl.ds(bb * SB, SB), :],
                    send_sem=send_sems.at[t, bb],
                    recv_sem=recv_sems.at[t, bb],
                    device_id=(right,),
                    device_id_type=pl.DeviceIdType.MESH,
                )
                rdma.start()
                rdmas.append(rdma)
            for rdma in rdmas:
                rdma.wait()

            b_acc = lax.rem(my + (N_DEV - 2 - t), N_DEV)
            last = t == n_hops - 1
            for bb in range(B):
                for nn in range(N // TN):
                    cols = pl.ds(nn * TN, TN)
                    cp_p = pltpu.make_async_copy(
                        p_ref.at[p_rows(b_acc, bb), cols], va, sem_a)
                    cp_r = pltpu.make_async_copy(
                        recv_ref.at[t, pl.ds(bb * SB, SB), cols], vb, sem_b)
                    cp_p.start()
                    cp_r.start()
                    cp_p.wait()
                    cp_r.wait()
                    vb[...] = va[...] + vb[...]
                    if last:
                        dst = out_ref.at[pl.ds(bb * SB, SB), cols]
                    else:
                        dst = recv_ref.at[t, pl.ds(bb * SB, SB), cols]
                    cp_o = pltpu.make_async_copy(vb, dst, sem_o)
                    cp_o.start()
                    cp_o.wait()

    return pl.pallas_call(
        body,
        out_shape=jax.ShapeDtypeStruct((B * SB, N), jnp.float32),
        in_specs=[pl.BlockSpec(memory_space=pltpu.ANY)],
        out_specs=pl.BlockSpec(memory_space=pltpu.ANY),
        scratch_shapes=[
            pltpu.ANY((n_hops, B * SB, N), jnp.float32),
            pltpu.VMEM((SB, TN), jnp.float32),
            pltpu.VMEM((SB, TN), jnp.float32),
            pltpu.SemaphoreType.DMA,
            pltpu.SemaphoreType.DMA,
            pltpu.SemaphoreType.DMA,
            pltpu.SemaphoreType.DMA((n_hops, B)),
            pltpu.SemaphoreType.DMA((n_hops, B)),
        ],
        compiler_params=pltpu.CompilerParams(collective_id=0),
    )(P)


def kernel(O, Wo):
    B, S, HL, D = O.shape
    K, N = Wo.shape
    assert HL * D == K
    A = O.reshape(B * S, K)
    P = _partial_matmul(A, Wo)
    SB = S // N_DEV
    out = _ring_reduce_scatter(P, B=B, S=S, SB=SB, N=N)
    return out.reshape(B, SB, N)


# baseline (device time: 2965177 ns/iter reference)
import jax
import jax.numpy as jnp
from jax import lax
from jax.experimental import pallas as pl
from jax.experimental.pallas import tpu as pltpu

N_DEV = 4


def _partial_matmul(A, Wo):
    M, K = A.shape
    _, N = Wo.shape
    bm, bn = 512, 1024

    def body(a_ref, w_ref, o_ref):
        o_ref[...] = jnp.dot(
            a_ref[...], w_ref[...], preferred_element_type=jnp.float32
        )

    return pl.pallas_call(
        body,
        grid=(M // bm, N // bn),
        in_specs=[
            pl.BlockSpec((bm, K), lambda i, j: (i, 0)),
            pl.BlockSpec((K, bn), lambda i, j: (0, j)),
        ],
        out_specs=pl.BlockSpec((bm, bn), lambda i, j: (i, j)),
        out_shape=jax.ShapeDtypeStruct((M, N), jnp.float32),
    )(A, Wo)


def _ring_reduce_scatter(P, B, S, SB, N):
    n_hops = N_DEV - 1
    TN = 4096

    def body(p_ref, out_ref, recv_ref, va, vb, sem_a, sem_b, sem_o,
             send_sems, recv_sems):
        my = lax.axis_index("i")
        left = lax.rem(my + N_DEV - 1, N_DEV)
        right = lax.rem(my + 1, N_DEV)

        barrier = pltpu.get_barrier_semaphore()
        for nbr in (left, right):
            pl.semaphore_signal(
                barrier, inc=1,
                device_id=(nbr,), device_id_type=pl.DeviceIdType.MESH,
            )
        pl.semaphore_wait(barrier, 2)

        def p_rows(b, bb):
            return pl.ds(bb * S + SB * b, SB)

        for t in range(n_hops):
            b_send = lax.rem(my + (N_DEV - 1 - t), N_DEV)
            rdmas = []
            for bb in range(B):
                if t == 0:
                    src = p_ref.at[p_rows(b_send, bb), :]
                else:
                    src = recv_ref.at[t - 1, pl.ds(bb * SB, SB), :]
                rdma = pltpu.make_async_remote_copy(
                    src_ref=src,
                    dst_ref=recv_ref.at[t, pl.ds(bb * SB, SB), :],
                    send_sem=send_sems.at[t, bb],
                    recv_sem=recv_sems.at[t, bb],
                    device_id=(right,),
                    device_id_type=pl.DeviceIdType.MESH,
                )
                rdma.start()
                rdmas.append(rdma)
            for rdma in rdmas:
                rdma.wait()

            b_acc = lax.rem(my + (N_DEV - 2 - t), N_DEV)
            last = t == n_hops - 1
            for bb in range(B):
                for nn in range(N // TN):
                    cols = pl.ds(nn * TN, TN)
                    cp_p = pltpu.make_async_copy(
                        p_ref.at[p_rows(b_acc, bb), cols], va, sem_a)
                    cp_r = pltpu.make_async_copy(
                        recv_ref.at[t, pl.ds(bb * SB, SB), cols], vb, sem_b)
                    cp_p.start()
                    cp_r.start()
                    cp_p.wait()
                    cp_r.wait()
                    vb[...] = va[...] + vb[...]
                    if last:
                        dst = out_ref.at[pl.ds(bb * SB, SB), cols]
                    else:
                        dst = recv_ref.at[t, pl.ds(bb * SB, SB), cols]
                    cp_o = pltpu.make_async_copy(vb, dst, sem_o)
                    cp_o.start()
                    cp_o.wait()

    out, _ = pl.pallas_call(
        body,
        out_shape=[
            jax.ShapeDtypeStruct((B * SB, N), jnp.float32),
            jax.ShapeDtypeStruct((n_hops, B * SB, N), jnp.float32),
        ],
        in_specs=[pl.BlockSpec(memory_space=pl.ANY)],
        out_specs=[
            pl.BlockSpec(memory_space=pl.ANY),
            pl.BlockSpec(memory_space=pl.ANY),
        ],
        scratch_shapes=[
            pltpu.VMEM((SB, TN), jnp.float32),
            pltpu.VMEM((SB, TN), jnp.float32),
            pltpu.SemaphoreType.DMA,
            pltpu.SemaphoreType.DMA,
            pltpu.SemaphoreType.DMA,
            pltpu.SemaphoreType.DMA((n_hops, B)),
            pltpu.SemaphoreType.DMA((n_hops, B)),
        ],
        compiler_params=pltpu.CompilerParams(collective_id=0),
    )(P)
    return out


def kernel(O, Wo):
    B, S, HL, D = O.shape
    K, N = Wo.shape
    assert HL * D == K
    A = O.reshape(B * S, K)
    P = _partial_matmul(A, Wo)
    SB = S // N_DEV
    out = _ring_reduce_scatter(P, B=B, S=S, SB=SB, N=N)
    return out.reshape(B, SB, N)


# device time: 2555234 ns/iter; 1.1604x vs baseline; 1.1604x over previous
import jax
import jax.numpy as jnp
from jax import lax
from jax.experimental import pallas as pl
from jax.experimental.pallas import tpu as pltpu

N_DEV = 4


def _fused_matmul_reduce_scatter(A, Wo, B, S, SB, N):
    M, K = A.shape
    n_hops = N_DEV - 1
    BN = 2048
    NN = N // BN
    RB = B * SB

    def body(a_ref, w_ref, out_ref, recv_ref, part_ref, send_sems, recv_sems):
        my = lax.axis_index("i")
        left = lax.rem(my + N_DEV - 1, N_DEV)
        right = lax.rem(my + 1, N_DEV)

        def compute_block(b, dst_ref):
            def mm(a_vr, w_vr, o_vr):
                o_vr[...] = jnp.dot(
                    a_vr[...], w_vr[...], preferred_element_type=jnp.float32
                )

            pltpu.emit_pipeline(
                mm,
                grid=(NN, B),
                in_specs=[
                    pl.BlockSpec((SB, K), lambda nn, bb: (bb * N_DEV + b, 0)),
                    pl.BlockSpec((K, BN), lambda nn, bb: (0, nn)),
                ],
                out_specs=[pl.BlockSpec((SB, BN), lambda nn, bb: (bb, nn))],
            )(a_ref, w_ref, dst_ref)

        def accumulate(src_ref, part_slot_ref, dst_ref):
            def add(r_vr, p_vr, o_vr):
                o_vr[...] = r_vr[...] + p_vr[...]

            pltpu.emit_pipeline(
                add,
                grid=(B, NN),
                in_specs=[
                    pl.BlockSpec((SB, BN), lambda bb, nn: (bb, nn)),
                    pl.BlockSpec((SB, BN), lambda bb, nn: (bb, nn)),
                ],
                out_specs=[pl.BlockSpec((SB, BN), lambda bb, nn: (bb, nn))],
            )(src_ref, part_slot_ref, dst_ref)

        def hop(t, src_ref):
            rdma = pltpu.make_async_remote_copy(
                src_ref=src_ref,
                dst_ref=recv_ref.at[t],
                send_sem=send_sems.at[t],
                recv_sem=recv_sems.at[t],
                device_id=(right,),
                device_id_type=pl.DeviceIdType.MESH,
            )
            rdma.start()
            return rdma

        b0 = lax.rem(my + 3, N_DEV)
        b1 = lax.rem(my + 2, N_DEV)
        b2 = lax.rem(my + 1, N_DEV)
        b3 = my

        compute_block(b0, part_ref.at[0])

        barrier = pltpu.get_barrier_semaphore()
        for nbr in (left, right):
            pl.semaphore_signal(
                barrier, inc=1,
                device_id=(nbr,), device_id_type=pl.DeviceIdType.MESH,
            )
        pl.semaphore_wait(barrier, 2)

        r0 = hop(0, part_ref.at[0])
        compute_block(b1, part_ref.at[1])
        r0.wait()
        accumulate(recv_ref.at[0], part_ref.at[1], recv_ref.at[0])

        r1 = hop(1, recv_ref.at[0])
        compute_block(b2, part_ref.at[0])
        r1.wait()
        accumulate(recv_ref.at[1], part_ref.at[0], recv_ref.at[1])

        r2 = hop(2, recv_ref.at[1])
        compute_block(b3, part_ref.at[1])
        r2.wait()
        accumulate(recv_ref.at[2], part_ref.at[1], out_ref)

    out, _, _ = pl.pallas_call(
        body,
        out_shape=[
            jax.ShapeDtypeStruct((RB, N), jnp.float32),
            jax.ShapeDtypeStruct((n_hops, RB, N), jnp.float32),
            jax.ShapeDtypeStruct((2, RB, N), jnp.float32),
        ],
        in_specs=[
            pl.BlockSpec(memory_space=pl.ANY),
            pl.BlockSpec(memory_space=pl.ANY),
        ],
        out_specs=[
            pl.BlockSpec(memory_space=pl.ANY),
            pl.BlockSpec(memory_space=pl.ANY),
            pl.BlockSpec(memory_space=pl.ANY),
        ],
        scratch_shapes=[
            pltpu.SemaphoreType.DMA((n_hops,)),
            pltpu.SemaphoreType.DMA((n_hops,)),
        ],
        compiler_params=pltpu.CompilerParams(
            collective_id=0,
            vmem_limit_bytes=100 * 1024 * 1024,
        ),
    )(A, Wo)
    return out


def kernel(O, Wo):
    B, S, HL, D = O.shape
    K, N = Wo.shape
    assert HL * D == K
    A = O.reshape(B * S, K)
    SB = S // N_DEV
    out = _fused_matmul_reduce_scatter(A, Wo, B=B, S=S, SB=SB, N=N)
    return out.reshape(B, SB, N)


# device time: 2307451 ns/iter; 1.2850x vs baseline; 1.1074x over previous
import jax
import jax.numpy as jnp
from jax import lax
from jax.experimental import pallas as pl
from jax.experimental.pallas import tpu as pltpu

N_DEV = 4


def _fused_matmul_reduce_scatter(A, Wo, B, S, SB, N):
    M, K = A.shape
    n_hops = N_DEV - 1
    BN = 2048
    NN = N // BN
    RB = B * SB

    def body(a_ref, w_ref, out_ref, recv_ref, part_ref, send_sems, recv_sems):
        my = lax.axis_index("i")
        left = lax.rem(my + N_DEV - 1, N_DEV)
        right = lax.rem(my + 1, N_DEV)

        def rows(bb):
            return pl.ds(bb * SB, SB)

        def compute_block(b, dst_ref):
            def mm(a_vr, w_vr, o_vr):
                o_vr[...] = jnp.dot(
                    a_vr[...], w_vr[...], preferred_element_type=jnp.float32
                )

            pltpu.emit_pipeline(
                mm,
                grid=(NN, B),
                in_specs=[
                    pl.BlockSpec((SB, K), lambda nn, bb: (bb * N_DEV + b, 0)),
                    pl.BlockSpec((K, BN), lambda nn, bb: (0, nn)),
                ],
                out_specs=[pl.BlockSpec((SB, BN), lambda nn, bb: (bb, nn))],
            )(a_ref, w_ref, dst_ref)

        def compute_chunk(b, bb, dst_chunk_ref):
            def mm(a_vr, w_vr, o_vr):
                o_vr[...] = jnp.dot(
                    a_vr[...], w_vr[...], preferred_element_type=jnp.float32
                )

            pltpu.emit_pipeline(
                mm,
                grid=(NN,),
                in_specs=[
                    pl.BlockSpec((SB, K), lambda nn: (bb * N_DEV + b, 0)),
                    pl.BlockSpec((K, BN), lambda nn: (0, nn)),
                ],
                out_specs=[pl.BlockSpec((SB, BN), lambda nn: (0, nn))],
            )(a_ref, w_ref, dst_chunk_ref)

        def accumulate_chunk(src_chunk_ref, part_chunk_ref, dst_chunk_ref):
            def add(r_vr, p_vr, o_vr):
                o_vr[...] = r_vr[...] + p_vr[...]

            pltpu.emit_pipeline(
                add,
                grid=(NN,),
                in_specs=[
                    pl.BlockSpec((SB, BN), lambda nn: (0, nn)),
                    pl.BlockSpec((SB, BN), lambda nn: (0, nn)),
                ],
                out_specs=[pl.BlockSpec((SB, BN), lambda nn: (0, nn))],
            )(src_chunk_ref, part_chunk_ref, dst_chunk_ref)

        def send_chunk(t, bb, src_ref):
            rdma = pltpu.make_async_remote_copy(
                src_ref=src_ref.at[rows(bb), :],
                dst_ref=recv_ref.at[t, rows(bb), :],
                send_sem=send_sems.at[t, bb],
                recv_sem=recv_sems.at[t, bb],
                device_id=(right,),
                device_id_type=pl.DeviceIdType.MESH,
            )
            rdma.start()
            return rdma

        b0 = lax.rem(my + 3, N_DEV)
        b1 = lax.rem(my + 2, N_DEV)
        b2 = lax.rem(my + 1, N_DEV)
        b3 = my

        barrier = pltpu.get_barrier_semaphore()
        for nbr in (left, right):
            pl.semaphore_signal(
                barrier, inc=1,
                device_id=(nbr,), device_id_type=pl.DeviceIdType.MESH,
            )
        pl.semaphore_wait(barrier, 2)

        hop_rdmas = []
        for bb in range(B):
            compute_chunk(b0, bb, part_ref.at[0, rows(bb), :])
            hop_rdmas.append(send_chunk(0, bb, part_ref.at[0]))

        compute_block(b1, part_ref.at[1])

        for t in range(n_hops):
            last = t == n_hops - 1
            next_rdmas = []
            for bb in range(B):
                hop_rdmas[bb].wait()
                if last:
                    accumulate_chunk(
                        recv_ref.at[t, rows(bb), :],
                        part_ref.at[(t + 1) % 2, rows(bb), :],
                        out_ref.at[rows(bb), :],
                    )
                else:
                    accumulate_chunk(
                        recv_ref.at[t, rows(bb), :],
                        part_ref.at[(t + 1) % 2, rows(bb), :],
                        recv_ref.at[t, rows(bb), :],
                    )
                    next_rdmas.append(send_chunk(t + 1, bb, recv_ref.at[t]))
            hop_rdmas = next_rdmas
            if not last:
                nxt = lax.rem(my + 1 - t, N_DEV)
                compute_block(nxt, part_ref.at[t % 2])

    out, _, _ = pl.pallas_call(
        body,
        out_shape=[
            jax.ShapeDtypeStruct((RB, N), jnp.float32),
            jax.ShapeDtypeStruct((n_hops, RB, N), jnp.float32),
            jax.ShapeDtypeStruct((2, RB, N), jnp.float32),
        ],
        in_specs=[
            pl.BlockSpec(memory_space=pl.ANY),
            pl.BlockSpec(memory_space=pl.ANY),
        ],
        out_specs=[
            pl.BlockSpec(memory_space=pl.ANY),
            pl.BlockSpec(memory_space=pl.ANY),
            pl.BlockSpec(memory_space=pl.ANY),
        ],
        scratch_shapes=[
            pltpu.SemaphoreType.DMA((n_hops, B)),
            pltpu.SemaphoreType.DMA((n_hops, B)),
        ],
        compiler_params=pltpu.CompilerParams(
            collective_id=0,
            vmem_limit_bytes=100 * 1024 * 1024,
        ),
    )(A, Wo)
    return out


def kernel(O, Wo):
    B, S, HL, D = O.shape
    K, N = Wo.shape
    assert HL * D == K
    A = O.reshape(B * S, K)
    SB = S // N_DEV
    out = _fused_matmul_reduce_scatter(A, Wo, B=B, S=S, SB=SB, N=N)
    return out.reshape(B, SB, N)


# device time: 1233223 ns/iter; 2.4044x vs baseline; 1.8711x over previous
import jax
import jax.numpy as jnp
from jax import lax
from jax.experimental import pallas as pl
from jax.experimental.pallas import tpu as pltpu

N_DEV = 4


def _fused_matmul_reduce_scatter(A, Wo, B, S, SB, N):
    M, K = A.shape
    n_hops = N_DEV - 1
    BN = 2048
    NN = N // BN
    NH = N // 2
    NNH = NH // BN
    RB = B * SB

    def body(a_ref, w_ref, out_ref, recv_ref, part_ref,
             send_sems_a, recv_sems_a, send_sems_b, recv_sems_b):
        my = lax.axis_index("i")
        left = lax.rem(my + N_DEV - 1, N_DEV)
        right = lax.rem(my + 1, N_DEV)

        def rows(bb):
            return pl.ds(bb * SB, SB)

        def mm(a_vr, w_vr, o_vr):
            o_vr[...] = jnp.dot(
                a_vr[...], w_vr[...], preferred_element_type=jnp.float32
            )

        def compute_block(b, dst_ref):
            pltpu.emit_pipeline(
                mm,
                grid=(NN, B),
                in_specs=[
                    pl.BlockSpec((SB, K), lambda nn, bb: (bb * N_DEV + b, 0)),
                    pl.BlockSpec((K, BN), lambda nn, bb: (0, nn)),
                ],
                out_specs=[pl.BlockSpec((SB, BN), lambda nn, bb: (bb, nn))],
            )(a_ref, w_ref, dst_ref)

        def compute_half_block(b, off, dst_ref):
            pltpu.emit_pipeline(
                mm,
                grid=(NNH, B),
                in_specs=[
                    pl.BlockSpec((SB, K), lambda nn, bb: (bb * N_DEV + b, 0)),
                    pl.BlockSpec((K, BN), lambda nn, bb: (0, nn)),
                ],
                out_specs=[pl.BlockSpec((SB, BN), lambda nn, bb: (bb, nn))],
            )(a_ref, w_ref.at[:, pl.ds(off, NH)],
              dst_ref.at[:, pl.ds(off, NH)])

        def compute_chunk_half(b, bb, off, dst_ref):
            pltpu.emit_pipeline(
                mm,
                grid=(NNH,),
                in_specs=[
                    pl.BlockSpec((SB, K), lambda nn: (bb * N_DEV + b, 0)),
                    pl.BlockSpec((K, BN), lambda nn: (0, nn)),
                ],
                out_specs=[pl.BlockSpec((SB, BN), lambda nn: (0, nn))],
            )(a_ref, w_ref.at[:, pl.ds(off, NH)],
              dst_ref.at[rows(bb), pl.ds(off, NH)])

        def accumulate_chunk(src_chunk_ref, part_chunk_ref, dst_chunk_ref):
            def add(r_vr, p_vr, o_vr):
                o_vr[...] = r_vr[...] + p_vr[...]

            pltpu.emit_pipeline(
                add,
                grid=(NN,),
                in_specs=[
                    pl.BlockSpec((SB, BN), lambda nn: (0, nn)),
                    pl.BlockSpec((SB, BN), lambda nn: (0, nn)),
                ],
                out_specs=[pl.BlockSpec((SB, BN), lambda nn: (0, nn))],
            )(src_chunk_ref, part_chunk_ref, dst_chunk_ref)

        def send_chunk_a(t, bb, src_ref):
            rdma = pltpu.make_async_remote_copy(
                src_ref=src_ref.at[rows(bb), pl.ds(0, NH)],
                dst_ref=recv_ref.at[t, rows(bb), pl.ds(0, NH)],
                send_sem=send_sems_a.at[t, bb],
                recv_sem=recv_sems_a.at[t, bb],
                device_id=(right,),
                device_id_type=pl.DeviceIdType.MESH,
            )
            rdma.start()
            return rdma

        def send_chunk_b(t, bb, src_ref):
            rdma = pltpu.make_async_remote_copy(
                src_ref=src_ref.at[rows(bb), pl.ds(NH, NH)],
                dst_ref=recv_ref.at[t, rows(bb), pl.ds(NH, NH)],
                send_sem=send_sems_b.at[t, bb],
                recv_sem=recv_sems_b.at[t, bb],
                device_id=(left,),
                device_id_type=pl.DeviceIdType.MESH,
            )
            rdma.start()
            return rdma

        b_cw0 = lax.rem(my + 3, N_DEV)
        b_ccw0 = lax.rem(my + 1, N_DEV)
        b_h0 = lax.rem(my + 2, N_DEV)
        b_h2 = my

        barrier = pltpu.get_barrier_semaphore()
        for nbr in (left, right):
            pl.semaphore_signal(
                barrier, inc=1,
                device_id=(nbr,), device_id_type=pl.DeviceIdType.MESH,
            )
        pl.semaphore_wait(barrier, 2)

        rdmas_a, rdmas_b = [], []
        for bb in range(B):
            compute_chunk_half(b_cw0, bb, 0, part_ref.at[0])
            compute_chunk_half(b_ccw0, bb, NH, part_ref.at[0])
            rdmas_a.append(send_chunk_a(0, bb, part_ref.at[0]))
            rdmas_b.append(send_chunk_b(0, bb, part_ref.at[0]))

        compute_block(b_h0, part_ref.at[1])

        for t in range(n_hops):
            last = t == n_hops - 1
            next_a, next_b = [], []
            for bb in range(B):
                rdmas_a[bb].wait()
                rdmas_b[bb].wait()
                dst = out_ref if last else recv_ref.at[t]
                accumulate_chunk(
                    recv_ref.at[t, rows(bb), :],
                    part_ref.at[(t + 1) % 2, rows(bb), :],
                    dst.at[rows(bb), :],
                )
                if not last:
                    next_a.append(send_chunk_a(t + 1, bb, recv_ref.at[t]))
                    next_b.append(send_chunk_b(t + 1, bb, recv_ref.at[t]))
            rdmas_a, rdmas_b = next_a, next_b
            if t == 0:
                compute_half_block(b_ccw0, 0, part_ref.at[0])
                compute_half_block(b_cw0, NH, part_ref.at[0])
            elif t == 1:
                compute_block(b_h2, part_ref.at[1])

    out, _, _ = pl.pallas_call(
        body,
        out_shape=[
            jax.ShapeDtypeStruct((RB, N), jnp.float32),
            jax.ShapeDtypeStruct((n_hops, RB, N), jnp.float32),
            jax.ShapeDtypeStruct((2, RB, N), jnp.float32),
        ],
        in_specs=[
            pl.BlockSpec(memory_space=pl.ANY),
            pl.BlockSpec(memory_space=pl.ANY),
        ],
        out_specs=[
            pl.BlockSpec(memory_space=pl.ANY),
            pl.BlockSpec(memory_space=pl.ANY),
            pl.BlockSpec(memory_space=pl.ANY),
        ],
        scratch_shapes=[
            pltpu.SemaphoreType.DMA((n_hops, B)),
            pltpu.SemaphoreType.DMA((n_hops, B)),
            pltpu.SemaphoreType.DMA((n_hops, B)),
            pltpu.SemaphoreType.DMA((n_hops, B)),
        ],
        compiler_params=pltpu.CompilerParams(
            collective_id=0,
            vmem_limit_bytes=100 * 1024 * 1024,
        ),
    )(A, Wo)
    return out


def kernel(O, Wo):
    B, S, HL, D = O.shape
    K, N = Wo.shape
    assert HL * D == K
    A = O.reshape(B * S, K)
    SB = S // N_DEV
    out = _fused_matmul_reduce_scatter(A, Wo, B=B, S=S, SB=SB, N=N)
    return out.reshape(B, SB, N)
